# baseline (speedup 1.0000x reference)
import jax
import jax.numpy as jnp
from jax.experimental import pallas as pl
from jax.experimental.pallas import tpu as pltpu

B = 2048
HIDDEN = 2048
NUM_GROUPS = 4
NUM_EXPERTS = 64
EPG = 16
BT = 512
NBLK = B // BT

NEG_INF = float("-inf")


def _router_kernel(x_ref, ew_ref, gw_ref,
                   all_ref, topk_ref, wts_ref, lvar_ref, ent_ref,
                   psum_ref, esum_ref, elog_ref, glog_ref):
    i = pl.program_id(0)
    slot = jax.lax.rem(i, 2)

    @pl.when(i < NBLK)
    def _matmuls():
        x = x_ref[...]
        elog_ref[slot] = jax.lax.dot_general(
            x, ew_ref[...], (((1,), (1,)), ((), ())),
            preferred_element_type=jnp.float32)
        glog_ref[slot] = jax.lax.dot_general(
            x, gw_ref[...], (((1,), (1,)), ((), ())),
            preferred_element_type=jnp.float32)

    @pl.when(i > 0)
    def _epilogue():
        step = i - 1
        pslot = 1 - slot
        elogits = elog_ref[pslot]
        glogits = glog_ref[pslot]

        g0 = glogits[:, 0:1]
        g1 = glogits[:, 1:2]
        g2 = glogits[:, 2:3]
        g3 = glogits[:, 3:4]
        p01 = g0 >= g1
        m01v = jnp.where(p01, g0, g1)
        m01i = jnp.where(p01, 0, 1)
        s01v = jnp.where(p01, g1, g0)
        s01i = jnp.where(p01, 1, 0)
        p23 = g2 >= g3
        m23v = jnp.where(p23, g2, g3)
        m23i = jnp.where(p23, 2, 3)
        s23v = jnp.where(p23, g3, g2)
        s23i = jnp.where(p23, 3, 2)
        pw = m01v >= m23v
        gv1 = jnp.where(pw, m01v, m23v)
        gi1 = jnp.where(pw, m01i, m23i)
        c2v = jnp.where(pw, s01v, m01v)
        c2i = jnp.where(pw, s01i, m01i)
        c3v = jnp.where(pw, m23v, s23v)
        c3i = jnp.where(pw, m23i, s23i)
        p2 = c2v >= c3v
        gv2 = jnp.where(p2, c2v, c3v)
        gi2 = jnp.where(p2, c2i, c3i)

        ge = jnp.exp(gv2 - gv1)
        denom = 1.0 + ge
        logw1 = jnp.log(1.0 / denom + 1e-8)
        logw2 = jnp.log(ge / denom + 1e-8)

        iota_e = jax.lax.broadcasted_iota(jnp.int32, (BT, NUM_EXPERTS), 1)
        gid = iota_e // EPG
        bias = jnp.where(gid == gi1, logw1,
                         jnp.where(gid == gi2, logw2, NEG_INF))
        all64 = elogits + bias
        all_ref[...] = all64

        iota_col = jax.lax.broadcasted_iota(
            jnp.int32, (NUM_EXPERTS, 1), 0).astype(jnp.bfloat16)
        ev1 = jnp.max(all64, axis=1, keepdims=True)
        eq1 = (all64 == ev1).astype(jnp.bfloat16)
        ei1f = jnp.dot(eq1, iota_col, preferred_element_type=jnp.float32)
        ei1 = ei1f.astype(jnp.int32)
        emasked = jnp.where(iota_e == ei1, NEG_INF, all64)
        ev2 = jnp.max(emasked, axis=1, keepdims=True)
        eq2 = (emasked == ev2).astype(jnp.bfloat16)
        ei2f = jnp.dot(eq2, iota_col, preferred_element_type=jnp.float32)
        ei2 = ei2f.astype(jnp.int32)
        topk_ref[...] = jnp.concatenate([ei1, ei2], axis=1)

        ee = jnp.exp(ev2 - ev1)
        wdenom = 1.0 + ee
        wts_ref[...] = jnp.concatenate([1.0 / wdenom, ee / wdenom], axis=1)

        ones_row = jnp.ones((1, BT), jnp.float32)
        z = all64 - ev1
        pexp = jnp.exp(z)
        zsel = jnp.where((gid == gi1) | (gid == gi2), z, 0.0)
        cat_rows = jnp.concatenate([pexp, pexp * zsel], axis=1)
        iota_r = jax.lax.broadcasted_iota(jnp.int32, (2 * NUM_EXPERTS, 2), 0)
        iota_c = jax.lax.broadcasted_iota(jnp.int32, (2 * NUM_EXPERTS, 2), 1)
        wblk = ((iota_r // NUM_EXPERTS) == iota_c).astype(jnp.float32)
        rsums = jnp.dot(cat_rows, wblk, preferred_element_type=jnp.float32)
        psum = rsums[:, 0:1]
        wzsum = rsums[:, 1:2]
        ent_row = jnp.log(psum) - wzsum / psum
        probs = pexp * (1.0 / psum)
        cat_cols = jnp.concatenate([probs, ent_row], axis=1)
        csums = jnp.dot(ones_row, cat_cols,
                        preferred_element_type=jnp.float32)
        probs_col = csums[:, :NUM_EXPERTS]
        ent_tot = csums[:, NUM_EXPERTS:NUM_EXPERTS + 1]

        @pl.when(step == 0)
        def _init():
            psum_ref[...] = probs_col
            esum_ref[...] = ent_tot

        @pl.when(step != 0)
        def _acc():
            psum_ref[...] += probs_col
            esum_ref[...] += ent_tot

        @pl.when(step == NBLK - 1)
        def _finalize():
            load = psum_ref[...] / jnp.float32(B)
            mu = jnp.mean(load)
            lvar_ref[...] = jnp.mean((load - mu) ** 2).reshape(1, 1)
            ent_ref[...] = esum_ref[...] / jnp.float32(B)


def kernel(hidden_states, group_router_weight, expert_router_weights):
    ewt = expert_router_weights.reshape(NUM_EXPERTS, HIDDEN)

    grid = (NBLK + 1,)
    out = pl.pallas_call(
        _router_kernel,
        grid=grid,
        in_specs=[
            pl.BlockSpec((BT, HIDDEN), lambda i: (jnp.minimum(i, NBLK - 1), 0)),
            pl.BlockSpec((NUM_EXPERTS, HIDDEN), lambda i: (0, 0)),
            pl.BlockSpec((NUM_GROUPS, HIDDEN), lambda i: (0, 0)),
        ],
        out_specs=[
            pl.BlockSpec((BT, NUM_EXPERTS), lambda i: (jnp.maximum(i - 1, 0), 0)),
            pl.BlockSpec((BT, 2), lambda i: (jnp.maximum(i - 1, 0), 0)),
            pl.BlockSpec((BT, 2), lambda i: (jnp.maximum(i - 1, 0), 0)),
            pl.BlockSpec((1, 1), lambda i: (0, 0)),
            pl.BlockSpec((1, 1), lambda i: (0, 0)),
        ],
        out_shape=[
            jax.ShapeDtypeStruct((B, NUM_EXPERTS), jnp.float32),
            jax.ShapeDtypeStruct((B, 2), jnp.int32),
            jax.ShapeDtypeStruct((B, 2), jnp.float32),
            jax.ShapeDtypeStruct((1, 1), jnp.float32),
            jax.ShapeDtypeStruct((1, 1), jnp.float32),
        ],
        scratch_shapes=[
            pltpu.VMEM((1, NUM_EXPERTS), jnp.float32),
            pltpu.VMEM((1, 1), jnp.float32),
            pltpu.VMEM((2, BT, NUM_EXPERTS), jnp.float32),
            pltpu.VMEM((2, BT, NUM_GROUPS), jnp.float32),
        ],
    )(hidden_states, ewt, group_router_weight)
    all64, topk, wts, lvar, ent = out
    return (all64, topk, wts, lvar.reshape(()), ent.reshape(()))


# speedup vs baseline: 1.0608x; 1.0608x over previous
import jax
import jax.numpy as jnp
from jax.experimental import pallas as pl
from jax.experimental.pallas import tpu as pltpu

B = 2048
HIDDEN = 2048
NUM_GROUPS = 4
NUM_EXPERTS = 64
EPG = 16
BT = 512
NBLK = B // BT

NEG_INF = float("-inf")


def _router_kernel(x_ref, ew_ref, gw_ref,
                   all_ref, topk_ref, wts_ref, lvar_ref, ent_ref,
                   psum_ref, esum_ref, elog_ref, glog_ref):
    i = pl.program_id(0)
    slot = jax.lax.rem(i, 2)

    @pl.when(i < NBLK)
    def _matmuls():
        x = x_ref[...]
        elog_ref[slot] = jax.lax.dot_general(
            x, ew_ref[...], (((1,), (1,)), ((), ())),
            preferred_element_type=jnp.float32)
        glog_ref[slot] = jax.lax.dot_general(
            x, gw_ref[...], (((1,), (1,)), ((), ())),
            preferred_element_type=jnp.float32)

    @pl.when(i > 0)
    def _epilogue():
        step = i - 1
        pslot = 1 - slot
        elogits = elog_ref[pslot]
        glogits = glog_ref[pslot]

        g0 = glogits[:, 0:1]
        g1 = glogits[:, 1:2]
        g2 = glogits[:, 2:3]
        g3 = glogits[:, 3:4]
        p01 = g0 >= g1
        m01v = jnp.where(p01, g0, g1)
        m01i = jnp.where(p01, 0, 1)
        s01v = jnp.where(p01, g1, g0)
        s01i = jnp.where(p01, 1, 0)
        p23 = g2 >= g3
        m23v = jnp.where(p23, g2, g3)
        m23i = jnp.where(p23, 2, 3)
        s23v = jnp.where(p23, g3, g2)
        s23i = jnp.where(p23, 3, 2)
        pw = m01v >= m23v
        gv1 = jnp.where(pw, m01v, m23v)
        gi1 = jnp.where(pw, m01i, m23i)
        c2v = jnp.where(pw, s01v, m01v)
        c2i = jnp.where(pw, s01i, m01i)
        c3v = jnp.where(pw, m23v, s23v)
        c3i = jnp.where(pw, m23i, s23i)
        p2 = c2v >= c3v
        gv2 = jnp.where(p2, c2v, c3v)
        gi2 = jnp.where(p2, c2i, c3i)

        ge = jnp.exp(gv2 - gv1)
        denom = 1.0 + ge
        logw1 = jnp.log(1.0 / denom + 1e-8)
        logw2 = jnp.log(ge / denom + 1e-8)

        iota_e = jax.lax.broadcasted_iota(jnp.int32, (BT, NUM_EXPERTS), 1)
        gid = iota_e // EPG
        bias = jnp.where(gid == gi1, logw1,
                         jnp.where(gid == gi2, logw2, NEG_INF))
        all64 = elogits + bias
        all_ref[...] = all64

        iota_col = jax.lax.broadcasted_iota(
            jnp.int32, (NUM_EXPERTS, 1), 0).astype(jnp.bfloat16)
        ev1 = jnp.max(all64, axis=1, keepdims=True)
        eq1 = (all64 == ev1).astype(jnp.bfloat16)
        ei1f = jnp.dot(eq1, iota_col, preferred_element_type=jnp.float32)
        ei1 = ei1f.astype(jnp.int32)
        emasked = jnp.where(iota_e == ei1, NEG_INF, all64)
        ev2 = jnp.max(emasked, axis=1, keepdims=True)
        eq2 = (emasked == ev2).astype(jnp.bfloat16)
        ei2f = jnp.dot(eq2, iota_col, preferred_element_type=jnp.float32)
        ei2 = ei2f.astype(jnp.int32)
        topk_ref[...] = jnp.concatenate([ei1, ei2], axis=1)

        ee = jnp.exp(ev2 - ev1)
        wdenom = 1.0 + ee
        wts_ref[...] = jnp.concatenate([1.0 / wdenom, ee / wdenom], axis=1)

        ones_col = jnp.ones((NUM_EXPERTS, 1), jnp.float32)
        ones_row = jnp.ones((1, BT), jnp.float32)
        z = all64 - ev1
        pexp = jnp.exp(z)
        psum = jnp.dot(pexp, ones_col, preferred_element_type=jnp.float32)
        zsel = jnp.where((gid == gi1) | (gid == gi2), z, 0.0)
        wzsum = jnp.dot(pexp * zsel, ones_col,
                        preferred_element_type=jnp.float32)
        ent_row = jnp.log(psum) - wzsum / psum
        probs = pexp * (1.0 / psum)
        probs_col = jnp.dot(ones_row, probs,
                            preferred_element_type=jnp.float32)
        ent_tot = jnp.dot(ones_row, ent_row,
                          preferred_element_type=jnp.float32)

        @pl.when(step == 0)
        def _init():
            psum_ref[...] = probs_col
            esum_ref[...] = ent_tot

        @pl.when(step != 0)
        def _acc():
            psum_ref[...] += probs_col
            esum_ref[...] += ent_tot

        @pl.when(step == NBLK - 1)
        def _finalize():
            load = psum_ref[...] / jnp.float32(B)
            mu = jnp.mean(load)
            lvar_ref[...] = jnp.mean((load - mu) ** 2).reshape(1, 1)
            ent_ref[...] = esum_ref[...] / jnp.float32(B)


def kernel(hidden_states, group_router_weight, expert_router_weights):
    ewt = expert_router_weights.reshape(NUM_EXPERTS, HIDDEN)

    grid = (NBLK + 1,)
    out = pl.pallas_call(
        _router_kernel,
        grid=grid,
        in_specs=[
            pl.BlockSpec((BT, HIDDEN), lambda i: (jnp.minimum(i, NBLK - 1), 0)),
            pl.BlockSpec((NUM_EXPERTS, HIDDEN), lambda i: (0, 0)),
            pl.BlockSpec((NUM_GROUPS, HIDDEN), lambda i: (0, 0)),
        ],
        out_specs=[
            pl.BlockSpec((BT, NUM_EXPERTS), lambda i: (jnp.maximum(i - 1, 0), 0)),
            pl.BlockSpec((BT, 2), lambda i: (jnp.maximum(i - 1, 0), 0)),
            pl.BlockSpec((BT, 2), lambda i: (jnp.maximum(i - 1, 0), 0)),
            pl.BlockSpec((1, 1), lambda i: (0, 0)),
            pl.BlockSpec((1, 1), lambda i: (0, 0)),
        ],
        out_shape=[
            jax.ShapeDtypeStruct((B, NUM_EXPERTS), jnp.float32),
            jax.ShapeDtypeStruct((B, 2), jnp.int32),
            jax.ShapeDtypeStruct((B, 2), jnp.float32),
            jax.ShapeDtypeStruct((1, 1), jnp.float32),
            jax.ShapeDtypeStruct((1, 1), jnp.float32),
        ],
        scratch_shapes=[
            pltpu.VMEM((1, NUM_EXPERTS), jnp.float32),
            pltpu.VMEM((1, 1), jnp.float32),
            pltpu.VMEM((2, BT, NUM_EXPERTS), jnp.float32),
            pltpu.VMEM((2, BT, NUM_GROUPS), jnp.float32),
        ],
    )(hidden_states, ewt, group_router_weight)
    all64, topk, wts, lvar, ent = out
    return (all64, topk, wts, lvar.reshape(()), ent.reshape(()))
